# Initial kernel scaffold; baseline (speedup 1.0000x reference)
import jax
import jax.numpy as jnp
from jax.experimental import pallas as pl


def kernel(x, norm1_w, norm2_w, Wqkv, Wout, router_W, W1, W2):
    raise NotImplementedError("write your pallas kernel here")


# speedup vs baseline: 2.0586x; 2.0586x over previous
import functools

import jax
import jax.numpy as jnp
from jax.experimental import pallas as pl
from jax.experimental.pallas import tpu as pltpu

EPS = 1.1920929e-07
LOG_BASE = 9.210340371976184
S = 2048
D = 768
H = 12
DK = 64
E = 8
F = 2048
TQ = 256
T1 = 256
TM = 256
NEG = -1e30

bf16 = jnp.bfloat16
f32 = jnp.float32


def _rope(x, base):
    n = x.shape[0]
    pos = base + jax.lax.broadcasted_iota(jnp.int32, (n, 32), 0).astype(f32)
    fidx = jax.lax.broadcasted_iota(jnp.int32, (n, 32), 1).astype(f32)
    inv = jnp.exp(fidx * (-LOG_BASE / 32.0))
    ang = pos * inv
    c = jnp.cos(ang)
    sn = jnp.sin(ang)
    x1 = x[:, :32]
    x2 = x[:, 32:]
    return jnp.concatenate([x1 * c - x2 * sn, x2 * c + x1 * sn], axis=-1)


def _gelu(x):
    return 0.5 * x * (1.0 + jax.lax.erf(x * 0.7071067811865476))


def _qkv_kernel(x_ref, n1_ref, w_ref, o_ref):
    x = x_ref[...]
    h = x * jax.lax.rsqrt(jnp.mean(x * x, axis=-1, keepdims=True) + EPS) * n1_ref[...]
    o_ref[...] = jax.lax.dot_general(
        h.astype(bf16), w_ref[...], (((1,), (1,)), ((), ())),
        preferred_element_type=f32).astype(bf16)


def _run_qkv(xs, n1, wqkv_bf):
    return pl.pallas_call(
        _qkv_kernel,
        grid=(S // T1,),
        in_specs=[
            pl.BlockSpec((T1, D), lambda i: (i, 0)),
            pl.BlockSpec((1, D), lambda i: (0, 0)),
            pl.BlockSpec((3 * D, D), lambda i: (0, 0)),
        ],
        out_specs=pl.BlockSpec((T1, 3 * D), lambda i: (i, 0)),
        out_shape=jax.ShapeDtypeStruct((S, 3 * D), bf16),
        compiler_params=pltpu.CompilerParams(
            dimension_semantics=("arbitrary",)),
    )(xs, n1, wqkv_bf)


def _attn_one(q, kr, v, qt):
    s = jax.lax.dot_general(q, kr, (((1,), (1,)), ((), ())),
                            preferred_element_type=f32) * 0.125
    row = qt * TQ + jax.lax.broadcasted_iota(jnp.int32, (TQ, S), 0)
    col = jax.lax.broadcasted_iota(jnp.int32, (TQ, S), 1)
    s = jnp.where(col <= row, s, NEG)
    m = jnp.max(s, axis=-1, keepdims=True)
    p = jnp.exp(s - m)
    p = p / jnp.sum(p, axis=-1, keepdims=True)
    return jax.lax.dot_general(p.astype(bf16), v, (((1,), (0,)), ((), ())),
                               preferred_element_type=f32)


def _attn_kernel(q_ref, k_ref, v_ref, o_ref, kr_ref):
    qt = pl.program_id(1)

    @pl.when(qt == 0)
    def _():
        k = k_ref[...].astype(f32)
        kr_ref[...] = jnp.concatenate(
            [_rope(k[:, :DK], 0.0), _rope(k[:, DK:], 0.0)],
            axis=-1).astype(bf16)

    qf = q_ref[...].astype(f32)
    base = jnp.float32(qt) * TQ
    kr = kr_ref[...]
    v = v_ref[...]
    ol = _attn_one(_rope(qf[:, :DK], base).astype(bf16), kr[:, :DK], v[:, :DK], qt)
    orr = _attn_one(_rope(qf[:, DK:], base).astype(bf16), kr[:, DK:], v[:, DK:], qt)
    o_ref[...] = jnp.concatenate([ol, orr], axis=-1).astype(bf16)


def _run_attn(qkv):
    return pl.pallas_call(
        _attn_kernel,
        grid=(H // 2, S // TQ),
        in_specs=[
            pl.BlockSpec((TQ, 2 * DK), lambda h, qt: (qt, h)),
            pl.BlockSpec((S, 2 * DK), lambda h, qt: (0, H // 2 + h)),
            pl.BlockSpec((S, 2 * DK), lambda h, qt: (0, H + h)),
        ],
        out_specs=pl.BlockSpec((TQ, 2 * DK), lambda h, qt: (qt, h)),
        out_shape=jax.ShapeDtypeStruct((S, D), bf16),
        scratch_shapes=[pltpu.VMEM((S, 2 * DK), bf16)],
        compiler_params=pltpu.CompilerParams(
            dimension_semantics=("arbitrary", "arbitrary")),
    )(qkv, qkv, qkv)


def _post_kernel(x_ref, ao_ref, wo_ref, n2_ref, rw_ref,
                 x2_ref, hf_ref, cw_ref, aux_ref, ps_ref):
    tt = pl.program_id(0)
    x2 = x_ref[...] + jax.lax.dot_general(
        ao_ref[...], wo_ref[...], (((1,), (1,)), ((), ())),
        preferred_element_type=f32)
    x2_ref[...] = x2
    hf = x2 * jax.lax.rsqrt(jnp.mean(x2 * x2, axis=-1, keepdims=True) + EPS) * n2_ref[...]
    hf_ref[...] = hf
    logits = jax.lax.dot_general(hf, rw_ref[...], (((1,), (1,)), ((), ())),
                                 preferred_element_type=f32)
    colf = jax.lax.broadcasted_iota(jnp.int32, (T1, 128), 1)
    logits = jnp.where(colf < E, logits, NEG)
    m = jnp.max(logits, axis=-1, keepdims=True)
    p = jnp.exp(logits - m)
    probs = p / jnp.sum(p, axis=-1, keepdims=True)

    @pl.when(tt == 0)
    def _():
        ps_ref[...] = jnp.zeros_like(ps_ref)

    ps_ref[...] += jnp.sum(probs, axis=0, keepdims=True)

    @pl.when(tt == pl.num_programs(0) - 1)
    def _():
        mp = ps_ref[...] / jnp.float32(S)
        aux_ref[...] = jnp.sum(mp * mp, axis=-1, keepdims=True) * jnp.float32(E)

    m1 = jnp.max(probs, axis=-1, keepdims=True)
    i1 = jnp.min(jnp.where(probs == m1, colf, 128), axis=-1, keepdims=True)
    probs2 = jnp.where(colf == i1, -1.0, probs)
    m2 = jnp.max(probs2, axis=-1, keepdims=True)
    i2 = jnp.min(jnp.where(probs2 == m2, colf, 128), axis=-1, keepdims=True)
    tot = m1 + m2
    w1 = m1 / tot
    w2 = m2 / tot
    cw_ref[...] = (jnp.where(colf == i1, w1, 0.0)
                   + jnp.where(colf == i2, w2, 0.0))


def _run_post(xs, ao, wout_bf, n2, rw_pad):
    return pl.pallas_call(
        _post_kernel,
        grid=(S // T1,),
        in_specs=[
            pl.BlockSpec((T1, D), lambda i: (i, 0)),
            pl.BlockSpec((T1, D), lambda i: (i, 0)),
            pl.BlockSpec((D, D), lambda i: (0, 0)),
            pl.BlockSpec((1, D), lambda i: (0, 0)),
            pl.BlockSpec((128, D), lambda i: (0, 0)),
        ],
        out_specs=[
            pl.BlockSpec((T1, D), lambda i: (i, 0)),
            pl.BlockSpec((T1, D), lambda i: (i, 0)),
            pl.BlockSpec((T1, 128), lambda i: (i, 0)),
            pl.BlockSpec((1, 1), lambda i: (0, 0)),
        ],
        out_shape=[
            jax.ShapeDtypeStruct((S, D), f32),
            jax.ShapeDtypeStruct((S, D), f32),
            jax.ShapeDtypeStruct((S, 128), f32),
            jax.ShapeDtypeStruct((1, 1), f32),
        ],
        scratch_shapes=[pltpu.VMEM((1, 128), f32)],
        compiler_params=pltpu.CompilerParams(
            dimension_semantics=("arbitrary",)),
    )(xs, ao, wout_bf, n2, rw_pad)


def _moe_kernel(hf_ref, cw_ref, x2_ref, w1_ref, w2_ref, o_ref):
    e = pl.program_id(0)
    tt = pl.program_id(1)
    rows = hf_ref[pl.ds(tt * TM, TM), :].astype(bf16)
    he = jax.lax.dot_general(rows, w1_ref[0], (((1,), (1,)), ((), ())),
                             preferred_element_type=f32)
    he = _gelu(he)
    oe = jax.lax.dot_general(he.astype(bf16), w2_ref[0], (((1,), (1,)), ((), ())),
                             preferred_element_type=f32)
    c = jnp.sum(
        jnp.where(jax.lax.broadcasted_iota(jnp.int32, (TM, 128), 1) == e,
                  cw_ref[pl.ds(tt * TM, TM), :], 0.0),
        axis=-1, keepdims=True)

    @pl.when(e == 0)
    def _():
        o_ref[pl.ds(tt * TM, TM), :] = x2_ref[pl.ds(tt * TM, TM), :] + c * oe

    @pl.when(e > 0)
    def _():
        o_ref[pl.ds(tt * TM, TM), :] += c * oe


def _run_moe(hf, cw, x2, w1_bf, w2_bf):
    return pl.pallas_call(
        _moe_kernel,
        grid=(E, S // TM),
        in_specs=[
            pl.BlockSpec((S, D), lambda e, t: (0, 0)),
            pl.BlockSpec((S, 128), lambda e, t: (0, 0)),
            pl.BlockSpec((S, D), lambda e, t: (0, 0)),
            pl.BlockSpec((1, F, D), lambda e, t: (e, 0, 0)),
            pl.BlockSpec((1, D, F), lambda e, t: (e, 0, 0)),
        ],
        out_specs=pl.BlockSpec((S, D), lambda e, t: (0, 0)),
        out_shape=jax.ShapeDtypeStruct((S, D), f32),
        compiler_params=pltpu.CompilerParams(
            dimension_semantics=("arbitrary", "arbitrary")),
    )(hf, cw, x2, w1_bf, w2_bf)


def kernel(x, norm1_w, norm2_w, Wqkv, Wout, router_W, W1, W2):
    xs = x.reshape(S, D)
    n1 = norm1_w.reshape(1, D)
    n2 = norm2_w.reshape(1, D)
    wqkv_bf = Wqkv.astype(bf16)
    wout_bf = Wout.astype(bf16)
    w1_bf = W1.astype(bf16)
    w2_bf = W2.astype(bf16)
    rw_pad = jnp.zeros((128, D), f32).at[:E].set(router_W)

    qkv = _run_qkv(xs, n1, wqkv_bf)
    ao = _run_attn(qkv)
    x2, hf, cw, aux = _run_post(xs, ao, wout_bf, n2, rw_pad)
    out = _run_moe(hf, cw, x2, w1_bf, w2_bf)
    return out.reshape(1, S, D), aux.reshape(())
